# baseline (speedup 1.0000x reference)
import functools

import jax
import jax.numpy as jnp
from jax import lax
from jax.experimental import pallas as pl
from jax.experimental.pallas import tpu as pltpu, tpu_sc as plsc

_NC = 2
_NS = 16
_NW = _NC * _NS
_K = 128


def _round_up(a, b):
    return (a + b - 1) // b * b


def _make_deg_kernel(RPW, NPAD):
    tile_n = NPAD // _NS
    mesh = plsc.VectorSubcoreMesh(core_axis_name="c", subcore_axis_name="s")

    @functools.partial(
        pl.kernel,
        mesh=mesh,
        out_type=jax.ShapeDtypeStruct((_NC * NPAD,), jnp.float32),
        scratch_types=[
            pltpu.VMEM((RPW, _K), jnp.int32),
            pltpu.VMEM((_K,), jnp.float32),
            pltpu.VMEM((tile_n,), jnp.float32),
            pltpu.VMEM_SHARED((NPAD,), jnp.float32),
        ],
    )
    def deg_kernel(dst_hbm, zeros_hbm, out_hbm, dst_v, ones_v, bnc_v, acc_sh):
        c = lax.axis_index("c")
        s = lax.axis_index("s")
        wid = s * _NC + c

        for i in range(_K // 16):
            ones_v[pl.ds(i * 16, 16)] = jnp.ones((16,), jnp.float32)

        sl = pl.ds(s * tile_n, tile_n)
        pltpu.sync_copy(zeros_hbm.at[sl], bnc_v)
        pltpu.sync_copy(bnc_v, acc_sh.at[sl])
        pltpu.sync_copy(dst_hbm.at[pl.ds(wid * RPW, RPW)], dst_v)
        plsc.subcore_barrier()

        def body(r, _):
            pltpu.sync_copy(ones_v, acc_sh.at[dst_v.at[r]], add=True)
            return 0

        lax.fori_loop(0, RPW, body, 0)
        plsc.subcore_barrier()

        pltpu.sync_copy(acc_sh.at[sl], bnc_v)
        pltpu.sync_copy(bnc_v, out_hbm.at[pl.ds(c * NPAD + s * tile_n, tile_n)])

    return deg_kernel


def _make_agg_kernel(RPW0, RPW1, NPAD, F):
    tile_n = NPAD // _NS
    mesh = plsc.VectorSubcoreMesh(core_axis_name="c", subcore_axis_name="s")
    NB = 8
    RPW = max(RPW0, RPW1)
    assert min(RPW0, RPW1) >= NB
    assert RPW0 % NB == 0 and RPW1 % NB == 0

    @functools.partial(
        pl.kernel,
        mesh=mesh,
        out_type=jax.ShapeDtypeStruct((_NC * NPAD, F), jnp.float32),
        scratch_types=[
            pltpu.VMEM((RPW, _K), jnp.int32),
            pltpu.VMEM((RPW, _K), jnp.int32),
            [pltpu.VMEM((_K, F), jnp.float32)] * NB,
            pltpu.VMEM((tile_n, F), jnp.float32),
            pltpu.VMEM_SHARED((NPAD, F), jnp.float32),
            [pltpu.SemaphoreType.DMA] * NB,
        ],
        compiler_params=pltpu.CompilerParams(use_tc_tiling_on_sc=False),
    )
    def agg_kernel(y_hbm, src_hbm, dst_hbm, zeros_hbm, out_hbm,
                   src_v, dst_v, rows, bnc_v, acc_sh, sems):
        c = lax.axis_index("c")
        s = lax.axis_index("s")

        sl = pl.ds(s * tile_n, tile_n)
        pltpu.sync_copy(zeros_hbm.at[sl], bnc_v)
        pltpu.sync_copy(bnc_v, acc_sh.at[sl])
        plsc.subcore_barrier()

        def gather(r, b):
            pltpu.async_copy(y_hbm.at[src_v.at[r]], rows[b], sems[b])

        def wait(b):
            pltpu.make_async_copy(y_hbm.at[src_v.at[0]], rows[b], sems[b]).wait()

        def scatter(r, b):
            pltpu.sync_copy(rows[b], acc_sh.at[dst_v.at[r]], add=True)

        def pipeline(start, n):
            pltpu.sync_copy(src_hbm.at[pl.ds(start, n)], src_v.at[pl.ds(0, n)])
            pltpu.sync_copy(dst_hbm.at[pl.ds(start, n)], dst_v.at[pl.ds(0, n)])

            for b in range(NB - 1):
                gather(b, b)

            def body(i, _):
                r = i * NB
                for b in range(NB):
                    wait(b)
                    gather(r + b + NB - 1, (b + NB - 1) % NB)
                    scatter(r + b, b)
                return 0

            lax.fori_loop(0, n // NB - 1, body, 0)
            r = n - NB
            gather(n - 1, NB - 1)
            for b in range(NB):
                wait(b)
                scatter(r + b, b)

        @pl.when(c == 0)
        def _():
            pipeline(s * RPW0, RPW0)

        @pl.when(c == 1)
        def _():
            pipeline(_NS * RPW0 + s * RPW1, RPW1)

        plsc.subcore_barrier()
        pltpu.sync_copy(acc_sh.at[sl], bnc_v)
        pltpu.sync_copy(bnc_v, out_hbm.at[pl.ds(c * NPAD + s * tile_n, tile_n)])

    return agg_kernel


def _make_t1(N, NPAD, HID):
    def t1_body(x_ref, w1_ref, dpair_ref, y1_ref, dinv_ref):
        deg = dpair_ref[0:N, :] + dpair_ref[NPAD:NPAD + N, :] + 1.0
        dinv = lax.rsqrt(deg)
        xw = jnp.dot(x_ref[...], w1_ref[...],
                     preferred_element_type=jnp.float32,
                     precision=lax.Precision.HIGHEST)
        y1_ref[0:N, :] = xw * dinv
        y1_ref[N:NPAD, :] = jnp.zeros((NPAD - N, HID), jnp.float32)
        dinv_ref[...] = dinv
    return t1_body


def _make_t2(N, NPAD, HID):
    def t2_body(a_ref, y1_ref, dinv_ref, b1_ref, g1_ref, be1_ref, z_ref):
        dinv = dinv_ref[...]
        h = (a_ref[0:N, :] + a_ref[NPAD:NPAD + N, :] + y1_ref[0:N, :]) * dinv \
            + b1_ref[...]
        mean = jnp.mean(h, axis=0, keepdims=True)
        cen = h - mean
        var = jnp.mean(cen * cen, axis=0, keepdims=True)
        hn = g1_ref[...] * cen / jnp.sqrt(var + 1e-5) + be1_ref[...]
        hr = jnp.maximum(hn, 0.0)
        z_ref[0:N, :] = hr * dinv
        z_ref[N:NPAD, :] = jnp.zeros((NPAD - N, HID), jnp.float32)
    return t2_body


def _make_t3(N, NPAD):
    def t3_body(a_ref, z_ref, dinv_ref, w2_ref, b2_ref, g2_ref, be2_ref,
                out_ref):
        w = a_ref[0:N, :] + a_ref[NPAD:NPAD + N, :] + z_ref[0:N, :]
        o = jnp.dot(w, w2_ref[...], preferred_element_type=jnp.float32,
                    precision=lax.Precision.HIGHEST) * dinv_ref[...] \
            + b2_ref[...]
        mean = jnp.mean(o, axis=0, keepdims=True)
        cen = o - mean
        var = jnp.mean(cen * cen, axis=0, keepdims=True)
        on = g2_ref[...] * cen / jnp.sqrt(var + 1e-5) + be2_ref[...]
        orl = jnp.maximum(on, 0.0)
        m = jnp.max(orl, axis=1, keepdims=True)
        e = jnp.exp(orl - m)
        out_ref[...] = (orl - m) - jnp.log(jnp.sum(e, axis=1, keepdims=True))
    return t3_body


def kernel(x, edge_index, W1, b1, gamma1, beta1, W2, b2, gamma2, beta2):
    N, F_IN = x.shape
    E = edge_index.shape[1]
    HID = W1.shape[1]
    C = W2.shape[1]
    assert HID % 16 == 0, "aggregated width must be a multiple of 16 f32"
    tile_n = _round_up((N + _NS - 1) // _NS, 128)
    NPAD = tile_n * _NS

    EP = _round_up(E, _K * _NW * 2)
    RPW = EP // (_K * _NW)
    src = edge_index[0]
    dst = edge_index[1]
    if EP != E:
        pad = jnp.full((EP - E,), N, dtype=jnp.int32)
        src = jnp.concatenate([src, pad])
        dst = jnp.concatenate([dst, pad])
    src2d = src.reshape(EP // _K, _K)
    dst2d = dst.reshape(EP // _K, _K)

    NCHUNK = EP // _K
    RPW0 = _round_up(int(NCHUNK * 0.2) // _NS, 8)
    RPW1 = NCHUNK // _NS - RPW0

    deg_kernel = _make_deg_kernel(RPW, NPAD)
    agg_h = _make_agg_kernel(RPW0, RPW1, NPAD, HID)

    zeros1 = jnp.zeros((NPAD,), jnp.float32)
    zeros_h = jnp.zeros((NPAD, HID), jnp.float32)

    deg_pair = deg_kernel(dst2d, zeros1).reshape(_NC * NPAD, 1)

    y1, dinv = pl.pallas_call(
        _make_t1(N, NPAD, HID),
        out_shape=[
            jax.ShapeDtypeStruct((NPAD, HID), jnp.float32),
            jax.ShapeDtypeStruct((N, 1), jnp.float32),
        ],
    )(x, W1, deg_pair)

    agg1 = agg_h(y1, src2d, dst2d, zeros_h)

    z = pl.pallas_call(
        _make_t2(N, NPAD, HID),
        out_shape=jax.ShapeDtypeStruct((NPAD, HID), jnp.float32),
    )(agg1, y1, dinv, b1.reshape(1, HID), gamma1.reshape(1, HID),
      beta1.reshape(1, HID))

    agg2 = agg_h(z, src2d, dst2d, zeros_h)

    out = pl.pallas_call(
        _make_t3(N, NPAD),
        out_shape=jax.ShapeDtypeStruct((N, C), jnp.float32),
    )(agg2, z, dinv, W2, b2.reshape(1, C), gamma2.reshape(1, C),
      beta2.reshape(1, C))

    return out


# speedup vs baseline: 1.0721x; 1.0721x over previous
import functools

import jax
import jax.numpy as jnp
from jax import lax
from jax.experimental import pallas as pl
from jax.experimental.pallas import tpu as pltpu, tpu_sc as plsc

_NC = 2
_NS = 16
_NW = _NC * _NS
_K = 128


def _round_up(a, b):
    return (a + b - 1) // b * b


def _make_deg_kernel(RPW, NPAD):
    tile_n = NPAD // _NS
    mesh = plsc.VectorSubcoreMesh(core_axis_name="c", subcore_axis_name="s")

    @functools.partial(
        pl.kernel,
        mesh=mesh,
        out_type=jax.ShapeDtypeStruct((_NC * NPAD,), jnp.float32),
        scratch_types=[
            pltpu.VMEM((RPW, _K), jnp.int32),
            pltpu.VMEM((_K,), jnp.float32),
            pltpu.VMEM((tile_n,), jnp.float32),
            pltpu.VMEM_SHARED((NPAD,), jnp.float32),
        ],
    )
    def deg_kernel(dst_hbm, zeros_hbm, out_hbm, dst_v, ones_v, bnc_v, acc_sh):
        c = lax.axis_index("c")
        s = lax.axis_index("s")
        wid = s * _NC + c

        for i in range(_K // 16):
            ones_v[pl.ds(i * 16, 16)] = jnp.ones((16,), jnp.float32)

        sl = pl.ds(s * tile_n, tile_n)
        pltpu.sync_copy(zeros_hbm.at[sl], bnc_v)
        pltpu.sync_copy(bnc_v, acc_sh.at[sl])
        pltpu.sync_copy(dst_hbm.at[pl.ds(wid * RPW, RPW)], dst_v)
        plsc.subcore_barrier()

        def body(r, _):
            pltpu.sync_copy(ones_v, acc_sh.at[dst_v.at[r]], add=True)
            return 0

        lax.fori_loop(0, RPW, body, 0)
        plsc.subcore_barrier()

        pltpu.sync_copy(acc_sh.at[sl], bnc_v)
        pltpu.sync_copy(bnc_v, out_hbm.at[pl.ds(c * NPAD + s * tile_n, tile_n)])

    return deg_kernel


def _make_agg_kernel(RPW0, RPW1, NPAD, F):
    tile_n = NPAD // _NS
    mesh = plsc.VectorSubcoreMesh(core_axis_name="c", subcore_axis_name="s")
    NB = 8
    RPW = max(RPW0, RPW1)
    assert min(RPW0, RPW1) >= NB
    assert RPW0 % NB == 0 and RPW1 % NB == 0

    @functools.partial(
        pl.kernel,
        mesh=mesh,
        out_type=jax.ShapeDtypeStruct((_NC * NPAD, F), jnp.float32),
        scratch_types=[
            pltpu.VMEM((RPW, _K), jnp.int32),
            pltpu.VMEM((RPW, _K), jnp.int32),
            [pltpu.VMEM((_K, F), jnp.float32)] * NB,
            pltpu.VMEM((tile_n, F), jnp.float32),
            pltpu.VMEM_SHARED((NPAD, F), jnp.float32),
            [pltpu.SemaphoreType.DMA] * NB,
        ],
        compiler_params=pltpu.CompilerParams(use_tc_tiling_on_sc=False),
    )
    def agg_kernel(y_hbm, src_hbm, dst_hbm, zeros_hbm, out_hbm,
                   src_v, dst_v, rows, bnc_v, acc_sh, sems):
        c = lax.axis_index("c")
        s = lax.axis_index("s")

        sl = pl.ds(s * tile_n, tile_n)
        pltpu.sync_copy(zeros_hbm.at[sl], bnc_v)
        pltpu.sync_copy(bnc_v, acc_sh.at[sl])
        plsc.subcore_barrier()

        def gather(r, b):
            pltpu.async_copy(y_hbm.at[src_v.at[r]], rows[b], sems[b])

        def wait(b):
            pltpu.make_async_copy(y_hbm.at[src_v.at[0]], rows[b], sems[b]).wait()

        def scatter(r, b):
            pltpu.sync_copy(rows[b], acc_sh.at[dst_v.at[r]], add=True)

        def pipeline(start, n):
            pltpu.sync_copy(src_hbm.at[pl.ds(start, n)], src_v.at[pl.ds(0, n)])
            pltpu.sync_copy(dst_hbm.at[pl.ds(start, n)], dst_v.at[pl.ds(0, n)])

            for b in range(NB - 1):
                gather(b, b)

            def body(i, _):
                r = i * NB
                for b in range(NB):
                    wait(b)
                    gather(r + b + NB - 1, (b + NB - 1) % NB)
                    scatter(r + b, b)
                return 0

            lax.fori_loop(0, n // NB - 1, body, 0)
            r = n - NB
            gather(n - 1, NB - 1)
            for b in range(NB):
                wait(b)
                scatter(r + b, b)

        @pl.when(c == 0)
        def _():
            pipeline(s * RPW0, RPW0)

        @pl.when(c == 1)
        def _():
            pipeline(_NS * RPW0 + s * RPW1, RPW1)

        plsc.subcore_barrier()
        pltpu.sync_copy(acc_sh.at[sl], bnc_v)
        pltpu.sync_copy(bnc_v, out_hbm.at[pl.ds(c * NPAD + s * tile_n, tile_n)])

    return agg_kernel


def _make_t1(N, NPAD, HID):
    def t1_body(x_ref, w1_ref, dpair_ref, y1_ref, dinv_ref):
        deg = dpair_ref[0:N, :] + dpair_ref[NPAD:NPAD + N, :] + 1.0
        dinv = lax.rsqrt(deg)
        xw = jnp.dot(x_ref[...], w1_ref[...],
                     preferred_element_type=jnp.float32,
                     precision=lax.Precision.HIGHEST)
        y1_ref[0:N, :] = xw * dinv
        y1_ref[N:NPAD, :] = jnp.zeros((NPAD - N, HID), jnp.float32)
        dinv_ref[...] = dinv
    return t1_body


def _make_t2(N, NPAD, HID):
    def t2_body(a_ref, y1_ref, dinv_ref, b1_ref, g1_ref, be1_ref, z_ref):
        dinv = dinv_ref[...]
        h = (a_ref[0:N, :] + a_ref[NPAD:NPAD + N, :] + y1_ref[0:N, :]) * dinv \
            + b1_ref[...]
        mean = jnp.mean(h, axis=0, keepdims=True)
        cen = h - mean
        var = jnp.mean(cen * cen, axis=0, keepdims=True)
        hn = g1_ref[...] * cen / jnp.sqrt(var + 1e-5) + be1_ref[...]
        hr = jnp.maximum(hn, 0.0)
        z_ref[0:N, :] = hr * dinv
        z_ref[N:NPAD, :] = jnp.zeros((NPAD - N, HID), jnp.float32)
    return t2_body


def _make_t3(N, NPAD):
    def t3_body(a_ref, z_ref, dinv_ref, w2_ref, b2_ref, g2_ref, be2_ref,
                out_ref):
        w = a_ref[0:N, :] + a_ref[NPAD:NPAD + N, :] + z_ref[0:N, :]
        o = jnp.dot(w, w2_ref[...], preferred_element_type=jnp.float32,
                    precision=lax.Precision.HIGHEST) * dinv_ref[...] \
            + b2_ref[...]
        mean = jnp.mean(o, axis=0, keepdims=True)
        cen = o - mean
        var = jnp.mean(cen * cen, axis=0, keepdims=True)
        on = g2_ref[...] * cen / jnp.sqrt(var + 1e-5) + be2_ref[...]
        orl = jnp.maximum(on, 0.0)
        m = jnp.max(orl, axis=1, keepdims=True)
        e = jnp.exp(orl - m)
        out_ref[...] = (orl - m) - jnp.log(jnp.sum(e, axis=1, keepdims=True))
    return t3_body


def kernel(x, edge_index, W1, b1, gamma1, beta1, W2, b2, gamma2, beta2):
    N, F_IN = x.shape
    E = edge_index.shape[1]
    HID = W1.shape[1]
    C = W2.shape[1]
    assert HID % 16 == 0, "aggregated width must be a multiple of 16 f32"
    tile_n = _round_up((N + _NS - 1) // _NS, 128)
    NPAD = tile_n * _NS

    EP = _round_up(E, _K * _NW * 2)
    RPW = EP // (_K * _NW)
    src = edge_index[0]
    dst = edge_index[1]
    if EP != E:
        pad = jnp.full((EP - E,), N, dtype=jnp.int32)
        src = jnp.concatenate([src, pad])
        dst = jnp.concatenate([dst, pad])
    src2d = src.reshape(EP // _K, _K)
    dst2d = dst.reshape(EP // _K, _K)

    NCHUNK = EP // _K
    RPW0 = _round_up(int(NCHUNK * 0.8) // _NS, 8)
    RPW1 = NCHUNK // _NS - RPW0

    deg_kernel = _make_deg_kernel(RPW, NPAD)
    agg_h = _make_agg_kernel(RPW0, RPW1, NPAD, HID)

    zeros1 = jnp.zeros((NPAD,), jnp.float32)
    zeros_h = jnp.zeros((NPAD, HID), jnp.float32)

    deg_pair = deg_kernel(dst2d, zeros1).reshape(_NC * NPAD, 1)

    y1, dinv = pl.pallas_call(
        _make_t1(N, NPAD, HID),
        out_shape=[
            jax.ShapeDtypeStruct((NPAD, HID), jnp.float32),
            jax.ShapeDtypeStruct((N, 1), jnp.float32),
        ],
    )(x, W1, deg_pair)

    agg1 = agg_h(y1, src2d, dst2d, zeros_h)

    z = pl.pallas_call(
        _make_t2(N, NPAD, HID),
        out_shape=jax.ShapeDtypeStruct((NPAD, HID), jnp.float32),
    )(agg1, y1, dinv, b1.reshape(1, HID), gamma1.reshape(1, HID),
      beta1.reshape(1, HID))

    agg2 = agg_h(z, src2d, dst2d, zeros_h)

    out = pl.pallas_call(
        _make_t3(N, NPAD),
        out_shape=jax.ShapeDtypeStruct((N, C), jnp.float32),
    )(agg2, z, dinv, W2, b2.reshape(1, C), gamma2.reshape(1, C),
      beta2.reshape(1, C))

    return out


# speedup vs baseline: 1.1533x; 1.0757x over previous
import functools

import jax
import jax.numpy as jnp
from jax import lax
from jax.experimental import pallas as pl
from jax.experimental.pallas import tpu as pltpu, tpu_sc as plsc

_NC = 2
_NS = 16
_NW = _NC * _NS
_K = 128


def _round_up(a, b):
    return (a + b - 1) // b * b


def _make_deg_kernel(RPW, NPAD):
    tile_n = NPAD // _NS
    mesh = plsc.VectorSubcoreMesh(core_axis_name="c", subcore_axis_name="s")

    @functools.partial(
        pl.kernel,
        mesh=mesh,
        out_type=jax.ShapeDtypeStruct((_NC * NPAD,), jnp.float32),
        scratch_types=[
            pltpu.VMEM((RPW, _K), jnp.int32),
            pltpu.VMEM((_K,), jnp.float32),
            pltpu.VMEM((tile_n,), jnp.float32),
            pltpu.VMEM_SHARED((NPAD,), jnp.float32),
        ],
    )
    def deg_kernel(ei_hbm, zeros_hbm, out_hbm, dst_v, ones_v, bnc_v, acc_sh):
        c = lax.axis_index("c")
        s = lax.axis_index("s")
        wid = s * _NC + c

        for i in range(_K // 16):
            ones_v[pl.ds(i * 16, 16)] = jnp.ones((16,), jnp.float32)

        sl = pl.ds(s * tile_n, tile_n)
        pltpu.sync_copy(zeros_hbm.at[sl], bnc_v)
        pltpu.sync_copy(bnc_v, acc_sh.at[sl])
        pltpu.sync_copy(ei_hbm.at[1, pl.ds(wid * RPW, RPW)], dst_v)
        plsc.subcore_barrier()

        def body(r, _):
            pltpu.sync_copy(ones_v, acc_sh.at[dst_v.at[r]], add=True)
            return 0

        lax.fori_loop(0, RPW, body, 0)
        plsc.subcore_barrier()

        pltpu.sync_copy(acc_sh.at[sl], bnc_v)
        pltpu.sync_copy(bnc_v, out_hbm.at[pl.ds(c * NPAD + s * tile_n, tile_n)])

    return deg_kernel


def _make_agg_kernel(RPW0, RPW1, NPAD, F):
    tile_n = NPAD // _NS
    mesh = plsc.VectorSubcoreMesh(core_axis_name="c", subcore_axis_name="s")
    NB = 8
    RPW = max(RPW0, RPW1)
    assert min(RPW0, RPW1) >= NB
    assert RPW0 % NB == 0 and RPW1 % NB == 0

    @functools.partial(
        pl.kernel,
        mesh=mesh,
        out_type=jax.ShapeDtypeStruct((_NC * NPAD, F), jnp.float32),
        scratch_types=[
            pltpu.VMEM((RPW, _K), jnp.int32),
            pltpu.VMEM((RPW, _K), jnp.int32),
            [pltpu.VMEM((_K, F), jnp.float32)] * NB,
            pltpu.VMEM((tile_n, F), jnp.float32),
            pltpu.VMEM_SHARED((NPAD, F), jnp.float32),
            [pltpu.SemaphoreType.DMA] * NB,
        ],
        compiler_params=pltpu.CompilerParams(use_tc_tiling_on_sc=False),
    )
    def agg_kernel(y_hbm, ei_hbm, zeros_hbm, out_hbm,
                   src_v, dst_v, rows, bnc_v, acc_sh, sems):
        c = lax.axis_index("c")
        s = lax.axis_index("s")

        sl = pl.ds(s * tile_n, tile_n)
        pltpu.sync_copy(zeros_hbm.at[sl], bnc_v)
        pltpu.sync_copy(bnc_v, acc_sh.at[sl])
        plsc.subcore_barrier()

        def gather(r, b):
            pltpu.async_copy(y_hbm.at[src_v.at[r]], rows[b], sems[b])

        def wait(b):
            pltpu.make_async_copy(y_hbm.at[src_v.at[0]], rows[b], sems[b]).wait()

        def scatter(r, b):
            pltpu.sync_copy(rows[b], acc_sh.at[dst_v.at[r]], add=True)

        def pipeline(start, n):
            pltpu.sync_copy(ei_hbm.at[0, pl.ds(start, n)],
                            src_v.at[pl.ds(0, n)])
            pltpu.sync_copy(ei_hbm.at[1, pl.ds(start, n)],
                            dst_v.at[pl.ds(0, n)])

            for b in range(NB - 1):
                gather(b, b)

            def body(i, _):
                r = i * NB
                for b in range(NB):
                    wait(b)
                    gather(r + b + NB - 1, (b + NB - 1) % NB)
                    scatter(r + b, b)
                return 0

            lax.fori_loop(0, n // NB - 1, body, 0)
            r = n - NB
            gather(n - 1, NB - 1)
            for b in range(NB):
                wait(b)
                scatter(r + b, b)

        @pl.when(c == 0)
        def _():
            pipeline(s * RPW0, RPW0)

        @pl.when(c == 1)
        def _():
            pipeline(_NS * RPW0 + s * RPW1, RPW1)

        plsc.subcore_barrier()
        pltpu.sync_copy(acc_sh.at[sl], bnc_v)
        pltpu.sync_copy(bnc_v, out_hbm.at[pl.ds(c * NPAD + s * tile_n, tile_n)])

    return agg_kernel


def _make_t1(N, NPAD, HID):
    def t1_body(x_ref, w1_ref, dpair_ref, y1_ref, dinv_ref):
        deg = dpair_ref[0:N, :] + dpair_ref[NPAD:NPAD + N, :] + 1.0
        dinv = lax.rsqrt(deg)
        xw = jnp.dot(x_ref[...], w1_ref[...],
                     preferred_element_type=jnp.float32,
                     precision=lax.Precision.HIGHEST)
        y1_ref[0:N, :] = xw * dinv
        y1_ref[N:NPAD, :] = jnp.zeros((NPAD - N, HID), jnp.float32)
        dinv_ref[...] = dinv
    return t1_body


def _make_t2(N, NPAD, HID):
    def t2_body(a_ref, y1_ref, dinv_ref, b1_ref, g1_ref, be1_ref, z_ref):
        dinv = dinv_ref[...]
        h = (a_ref[0:N, :] + a_ref[NPAD:NPAD + N, :] + y1_ref[0:N, :]) * dinv \
            + b1_ref[...]
        mean = jnp.mean(h, axis=0, keepdims=True)
        cen = h - mean
        var = jnp.mean(cen * cen, axis=0, keepdims=True)
        hn = g1_ref[...] * cen / jnp.sqrt(var + 1e-5) + be1_ref[...]
        hr = jnp.maximum(hn, 0.0)
        z_ref[0:N, :] = hr * dinv
        z_ref[N:NPAD, :] = jnp.zeros((NPAD - N, HID), jnp.float32)
    return t2_body


def _make_t3(N, NPAD):
    def t3_body(a_ref, z_ref, dinv_ref, w2_ref, b2_ref, g2_ref, be2_ref,
                out_ref):
        w = a_ref[0:N, :] + a_ref[NPAD:NPAD + N, :] + z_ref[0:N, :]
        o = jnp.dot(w, w2_ref[...], preferred_element_type=jnp.float32,
                    precision=lax.Precision.HIGHEST) * dinv_ref[...] \
            + b2_ref[...]
        mean = jnp.mean(o, axis=0, keepdims=True)
        cen = o - mean
        var = jnp.mean(cen * cen, axis=0, keepdims=True)
        on = g2_ref[...] * cen / jnp.sqrt(var + 1e-5) + be2_ref[...]
        orl = jnp.maximum(on, 0.0)
        m = jnp.max(orl, axis=1, keepdims=True)
        e = jnp.exp(orl - m)
        out_ref[...] = (orl - m) - jnp.log(jnp.sum(e, axis=1, keepdims=True))
    return t3_body


def kernel(x, edge_index, W1, b1, gamma1, beta1, W2, b2, gamma2, beta2):
    N, F_IN = x.shape
    E = edge_index.shape[1]
    HID = W1.shape[1]
    C = W2.shape[1]
    assert HID % 16 == 0, "aggregated width must be a multiple of 16 f32"
    tile_n = _round_up((N + _NS - 1) // _NS, 128)
    NPAD = tile_n * _NS

    EP = _round_up(E, _K * _NW * 2)
    RPW = EP // (_K * _NW)
    ei3 = edge_index
    if EP != E:
        ei3 = jnp.pad(edge_index, ((0, 0), (0, EP - E)), constant_values=N)
    ei3 = ei3.reshape(2, EP // _K, _K)

    NCHUNK = EP // _K
    RPW0 = _round_up(int(NCHUNK * 0.8) // _NS, 8)
    RPW1 = NCHUNK // _NS - RPW0

    deg_kernel = _make_deg_kernel(RPW, NPAD)
    agg_h = _make_agg_kernel(RPW0, RPW1, NPAD, HID)

    zeros1 = jnp.zeros((NPAD,), jnp.float32)
    zeros_h = jnp.zeros((NPAD, HID), jnp.float32)

    deg_pair = deg_kernel(ei3, zeros1).reshape(_NC * NPAD, 1)

    y1, dinv = pl.pallas_call(
        _make_t1(N, NPAD, HID),
        out_shape=[
            jax.ShapeDtypeStruct((NPAD, HID), jnp.float32),
            jax.ShapeDtypeStruct((N, 1), jnp.float32),
        ],
    )(x, W1, deg_pair)

    agg1 = agg_h(y1, ei3, zeros_h)

    z = pl.pallas_call(
        _make_t2(N, NPAD, HID),
        out_shape=jax.ShapeDtypeStruct((NPAD, HID), jnp.float32),
    )(agg1, y1, dinv, b1.reshape(1, HID), gamma1.reshape(1, HID),
      beta1.reshape(1, HID))

    agg2 = agg_h(z, ei3, zeros_h)

    out = pl.pallas_call(
        _make_t3(N, NPAD),
        out_shape=jax.ShapeDtypeStruct((N, C), jnp.float32),
    )(agg2, z, dinv, W2, b2.reshape(1, C), gamma2.reshape(1, C),
      beta2.reshape(1, C))

    return out
